# Initial kernel scaffold; baseline (speedup 1.0000x reference)
import jax
import jax.numpy as jnp
from jax.experimental import pallas as pl


def kernel(table, labels):
    raise NotImplementedError("write your pallas kernel here")


# speedup vs baseline: 1.8264x; 1.8264x over previous
import functools

import jax
import jax.numpy as jnp
from jax import lax
from jax.experimental import pallas as pl
from jax.experimental.pallas import tpu as pltpu
from jax.experimental.pallas import tpu_sc as plsc

CAPACITY = 64
NUM_CLUSTERS = 64
REG = 0.1


def _centers_body(table_ref, labels_ref, centers_ref):
    rows, dim = table_ref.shape
    lab = labels_ref[...]
    k_iota = lax.broadcasted_iota(jnp.int32, (rows, NUM_CLUSTERS), 1)
    onehot = (lab == k_iota).astype(jnp.float32)
    dn = (((0,), (0,)), ((), ()))
    sums = lax.dot_general(onehot, table_ref[...], dn,
                           preferred_element_type=jnp.float32,
                           precision=lax.Precision.HIGHEST)
    ones = jnp.ones((rows, 1), jnp.float32)
    counts = lax.dot_general(onehot, ones, dn,
                             preferred_element_type=jnp.float32,
                             precision=lax.Precision.HIGHEST)
    centers_ref[...] = sums / jnp.maximum(counts, 1.0)


def _centers_tc(table2d, labels_col):
    return pl.pallas_call(
        _centers_body,
        out_shape=jax.ShapeDtypeStruct((NUM_CLUSTERS, table2d.shape[1]),
                                       jnp.float32),
    )(table2d, labels_col)


def _gather_centers_sc(centers, labels2d, rows, dim):
    ncores, nsub = 2, 16
    nw = ncores * nsub
    rows_per_w = rows // nw

    @functools.partial(
        pl.kernel,
        out_type=jax.ShapeDtypeStruct((rows, dim), jnp.float32),
        mesh=plsc.VectorSubcoreMesh(core_axis_name="c", subcore_axis_name="s"),
        scratch_types=[
            pltpu.VMEM((rows_per_w,), jnp.int32),
            pltpu.VMEM((rows_per_w, dim), jnp.float32),
            pltpu.SemaphoreType.DMA,
        ],
    )
    def gather(centers_hbm, labels_hbm, out_hbm, lab_v, rows_v, sem):
        c = lax.axis_index("c")
        s = lax.axis_index("s")
        wid = s * ncores + c
        row0 = wid * rows_per_w
        pltpu.sync_copy(labels_hbm.at[wid], lab_v)
        pltpu.async_copy(centers_hbm.at[lab_v], rows_v, sem).wait()
        pltpu.sync_copy(rows_v, out_hbm.at[pl.ds(row0, rows_per_w)])

    return gather(centers, labels2d)


def _dist_body(table_ref, cdata_ref, d_ref):
    diff = table_ref[...] - cdata_ref[...]
    d_ref[...] = jnp.sum(diff * diff, axis=1, keepdims=True)


def _distances_tc(table2d, cdata):
    rows, dim = table2d.shape
    blk = 512
    return pl.pallas_call(
        _dist_body,
        grid=(rows // blk,),
        in_specs=[
            pl.BlockSpec((blk, dim), lambda i: (i, 0)),
            pl.BlockSpec((blk, dim), lambda i: (i, 0)),
        ],
        out_specs=pl.BlockSpec((blk, 1), lambda i: (i, 0)),
        out_shape=jax.ShapeDtypeStruct((rows, 1), jnp.float32),
    )(table2d, cdata)


_BLK = 512


def _ranks_body(dc_ref, lc_ref, dr_ref, lr_ref, soft_ref, ridx_ref,
                scores_ref):
    i = pl.program_id(0)
    d_blk = dc_ref[...]
    lab_blk = lc_ref[...].astype(jnp.float32)
    d_all = dr_ref[...]
    lab_all = lr_ref[...].astype(jnp.float32)
    rows = d_all.shape[1]
    mn = jnp.min(d_all)
    mx = jnp.max(d_all)
    s_all = (d_all - mn) / (mx - mn) + lab_all
    s_blk = (d_blk - mn) / (mx - mn) + lab_blk
    mn2 = jnp.min(s_all)
    mx2 = jnp.max(s_all)
    z_all = (s_all - mn2) / (mx2 - mn2)
    z_blk = (s_blk - mn2) / (mx2 - mn2)
    z = (z_blk - z_all) * jnp.float32(1.0 / REG)
    sig = 1.0 / (1.0 + jnp.exp(-z))
    soft_ref[...] = 0.5 + jnp.sum(sig, axis=1, keepdims=True)
    lt = (s_all < s_blk).astype(jnp.int32)
    j_iota = lax.broadcasted_iota(jnp.int32, (_BLK, rows), 1)
    i_idx = i * _BLK + lax.broadcasted_iota(jnp.int32, (_BLK, 1), 0)
    tie = jnp.where((s_all == s_blk) & (j_iota < i_idx), 1, 0)
    cnt = jnp.sum(lt + tie, axis=1, keepdims=True)
    ridx_ref[...] = cnt // CAPACITY + 1
    scores_ref[...] = s_blk


def _ranks_tc(d_col, labels_col, d_row, labels_row):
    rows = d_row.shape[1]
    grid = rows // _BLK
    return pl.pallas_call(
        _ranks_body,
        grid=(grid,),
        in_specs=[
            pl.BlockSpec((_BLK, 1), lambda i: (i, 0)),
            pl.BlockSpec((_BLK, 1), lambda i: (i, 0)),
            pl.BlockSpec((1, rows), lambda i: (0, 0)),
            pl.BlockSpec((1, rows), lambda i: (0, 0)),
        ],
        out_specs=[
            pl.BlockSpec((_BLK, 1), lambda i: (i, 0)),
            pl.BlockSpec((_BLK, 1), lambda i: (i, 0)),
            pl.BlockSpec((_BLK, 1), lambda i: (i, 0)),
        ],
        out_shape=[
            jax.ShapeDtypeStruct((rows, 1), jnp.float32),
            jax.ShapeDtypeStruct((rows, 1), jnp.int32),
            jax.ShapeDtypeStruct((rows, 1), jnp.float32),
        ],
    )(d_col, labels_col, d_row, labels_row)


def kernel(table, labels):
    rows = table.shape[1]
    dim = table.shape[-1]
    table2d = table.reshape(rows, dim)
    labels_col = labels.reshape(rows, 1)
    centers = _centers_tc(table2d, labels_col)
    cdata = _gather_centers_sc(centers, labels.reshape(rows // 128, 128),
                               rows, dim)
    d = _distances_tc(table2d, cdata)
    soft, ridx, scores = _ranks_tc(d, labels_col,
                                   d.reshape(1, rows), labels.reshape(1, rows))
    return (soft.reshape(1, rows, 1),
            ridx.reshape(1, rows, 1),
            scores.reshape(1, rows, 1))
